# baseline (speedup 1.0000x reference)
import functools

import jax
import jax.numpy as jnp
from jax.experimental import pallas as pl


def _mm_body(a_ref, w_ref, b_ref, o_ref, *, relu):
    acc = jnp.dot(a_ref[...], w_ref[...], preferred_element_type=jnp.float32)
    acc = acc + b_ref[...]
    if relu:
        acc = jnp.maximum(acc, 0.0)
    o_ref[...] = acc


def _mm(a, w, b, relu, bm=512):
    M, K = a.shape
    N = w.shape[1]
    bm = min(bm, max(8, M))
    Mp = -(-M // bm) * bm
    if Mp != M:
        a = jnp.pad(a, ((0, Mp - M), (0, 0)))
    bn = 128 if N > 128 else N
    out = pl.pallas_call(
        functools.partial(_mm_body, relu=relu),
        grid=(Mp // bm, N // bn),
        in_specs=[
            pl.BlockSpec((bm, K), lambda i, j: (i, 0)),
            pl.BlockSpec((K, bn), lambda i, j: (0, j)),
            pl.BlockSpec((1, bn), lambda i, j: (0, j)),
        ],
        out_specs=pl.BlockSpec((bm, bn), lambda i, j: (i, j)),
        out_shape=jax.ShapeDtypeStruct((Mp, N), jnp.float32),
    )(a, w, b.reshape(1, N))
    return out[:M]


def _im2col(x, stride, ho):
    n, _, _, c = x.shape
    span = stride * (ho - 1) + 1
    taps = []
    for dy in range(3):
        for dx in range(3):
            taps.append(x[:, dy:dy + span:stride, dx:dx + span:stride, :])
    p = jnp.stack(taps, axis=3)
    return p.reshape(n * ho * ho, 9 * c)


def _wflat(w):
    return jnp.transpose(w, (2, 3, 1, 0)).reshape(-1, w.shape[0])


def _conv_relu(x, w, b, stride, ho):
    n = x.shape[0]
    patches = _im2col(x, stride, ho)
    out = _mm(patches, _wflat(w), b, relu=True)
    return out.reshape(n, ho, ho, w.shape[0])


def _encoder(x_nchw, p, pre):
    x = jnp.transpose(x_nchw, (0, 2, 3, 1))
    x = _conv_relu(x, p[pre + '_w1'], p[pre + '_b1'], 2, 139)
    x = _conv_relu(x, p[pre + '_w2'], p[pre + '_b2'], 1, 137)
    x = _conv_relu(x, p[pre + '_w3'], p[pre + '_b3'], 2, 68)
    return jnp.transpose(x, (0, 3, 1, 2))


def _affine_grid(theta, hw):
    h = w = hw
    xs = (2.0 * jnp.arange(w, dtype=jnp.float32) + 1.0) / w - 1.0
    ys = (2.0 * jnp.arange(h, dtype=jnp.float32) + 1.0) / h - 1.0
    gx, gy = jnp.meshgrid(xs, ys)
    base = jnp.stack([gx, gy, jnp.ones_like(gx)], axis=-1)
    return jnp.einsum('nij,hwj->nhwi', theta, base)


def _grid_sample(im, grid):
    n, c, h, w = im.shape
    ix = ((grid[..., 0] + 1.0) * w - 1.0) / 2.0
    iy = ((grid[..., 1] + 1.0) * h - 1.0) / 2.0
    ix0 = jnp.floor(ix)
    iy0 = jnp.floor(iy)
    ix1 = ix0 + 1.0
    iy1 = iy0 + 1.0
    wx1 = ix - ix0
    wx0 = 1.0 - wx1
    wy1 = iy - iy0
    wy0 = 1.0 - wy1
    bidx = jnp.arange(n)[:, None, None]

    def g(xc, yc):
        xi = jnp.clip(xc, 0, w - 1).astype(jnp.int32)
        yi = jnp.clip(yc, 0, h - 1).astype(jnp.int32)
        valid = ((xc >= 0) & (xc <= w - 1) & (yc >= 0) & (yc <= h - 1))
        return im[bidx, :, yi, xi] * valid[..., None].astype(im.dtype)

    out = (g(ix0, iy0) * (wx0 * wy0)[..., None]
           + g(ix1, iy0) * (wx1 * wy0)[..., None]
           + g(ix0, iy1) * (wx0 * wy1)[..., None]
           + g(ix1, iy1) * (wx1 * wy1)[..., None])
    return jnp.transpose(out, (0, 3, 1, 2))


def _head_body(xw_ref, sp_ref, g1b, w2, b2, w3, b3, m1w, m1b, m2w, m2b,
               f1a, f1b_, f1bias, f2w, f2b, f3w, f3b, o_ref):
    xw = xw_ref[...]
    m = jnp.maximum(jnp.maximum(xw[0:4], xw[4:8]),
                    jnp.maximum(xw[8:12], xw[12:16]))
    z = jnp.maximum(0.25 * m + g1b[...], 0.0)
    z = jnp.maximum(0.25 * jnp.dot(z, w2[...],
                                   preferred_element_type=jnp.float32)
                    + b2[...], 0.0)
    z = 0.25 * jnp.dot(z, w3[...], preferred_element_type=jnp.float32) + b3[...]
    mt = jnp.maximum(sp_ref[...] * m1w[...] + m1b[...], 0.0)
    mt = jnp.maximum(jnp.dot(mt, m2w[...],
                             preferred_element_type=jnp.float32) + m2b[...], 0.0)
    h = jnp.maximum(jnp.dot(z, f1a[...], preferred_element_type=jnp.float32)
                    + jnp.dot(mt, f1b_[...], preferred_element_type=jnp.float32)
                    + f1bias[...], 0.0)
    h = jnp.maximum(jnp.dot(h, f2w[...], preferred_element_type=jnp.float32)
                    + f2b[...], 0.0)
    o_ref[...] = (jnp.dot(h, f3w[...], preferred_element_type=jnp.float32)
                  + f3b[...])


def _head(xw, speed, p):
    f1a = p['f1_w'][:512]
    f1b_ = p['f1_w'][512:]
    f3w = jnp.pad(p['f3_w'], ((0, 0), (0, 125)))
    f3b = jnp.pad(p['f3_b'], (0, 125))
    args = [xw, speed.reshape(4, 1),
            p['g1_b'].reshape(1, -1),
            p['g2_w'], p['g2_b'].reshape(1, -1),
            p['g3_w'], p['g3_b'].reshape(1, -1),
            p['m1_w'], p['m1_b'].reshape(1, -1),
            p['m2_w'], p['m2_b'].reshape(1, -1),
            f1a, f1b_, p['f1_b'].reshape(1, -1),
            p['f2_w'], p['f2_b'].reshape(1, -1),
            f3w, f3b.reshape(1, -1)]
    out = pl.pallas_call(
        _head_body,
        out_shape=jax.ShapeDtypeStruct((4, 128), jnp.float32),
    )(*args)
    return out[:, :3]


def kernel(ego_lidar, ego_speed, other_lidar, other_transform, params):
    B = ego_lidar.shape[0]
    ego_rep = _encoder(ego_lidar, params, 'e1')
    ol = jnp.transpose(other_lidar, (1, 0, 2, 3, 4)).reshape(
        3 * B, *other_lidar.shape[2:])
    other_rep = _encoder(ol, params, 'e2')

    sel = jnp.array([0, 1, 3])
    feats = [ego_rep.reshape(B, -1)]
    for i in range(3):
        rep = other_rep[i * B:(i + 1) * B]
        theta = other_transform[:, i][:, :2, :][:, :, sel]
        grid = _affine_grid(theta, 68)
        warped = _grid_sample(rep, grid)
        feats.append(warped.reshape(B, -1))
    x_all = jnp.concatenate(feats, axis=0)

    zeros512 = jnp.zeros((512,), jnp.float32)
    xw = _mm(x_all, params['g1_w'], zeros512, relu=False)

    pc = _head(xw, ego_speed, params)
    return (pc[:, 0], pc[:, 1], pc[:, 2])


# speedup vs baseline: 1.5750x; 1.5750x over previous
import functools

import jax
import jax.numpy as jnp
from jax.experimental import pallas as pl


def _mm_body(a_ref, w_ref, b_ref, o_ref, *, relu):
    acc = jnp.dot(a_ref[...], w_ref[...], preferred_element_type=jnp.float32)
    acc = acc + b_ref[...]
    if relu:
        acc = jnp.maximum(acc, 0.0)
    o_ref[...] = acc


def _mm(a, w, b, relu, bm=512):
    M, K = a.shape
    N = w.shape[1]
    bm = min(bm, max(8, M))
    Mp = -(-M // bm) * bm
    if Mp != M:
        a = jnp.pad(a, ((0, Mp - M), (0, 0)))
    bn = 128 if N > 128 else N
    out = pl.pallas_call(
        functools.partial(_mm_body, relu=relu),
        grid=(Mp // bm, N // bn),
        in_specs=[
            pl.BlockSpec((bm, K), lambda i, j: (i, 0)),
            pl.BlockSpec((K, bn), lambda i, j: (0, j)),
            pl.BlockSpec((1, bn), lambda i, j: (0, j)),
        ],
        out_specs=pl.BlockSpec((bm, bn), lambda i, j: (i, j)),
        out_shape=jax.ShapeDtypeStruct((Mp, N), jnp.float32),
    )(a, w, b.reshape(1, N))
    return out[:M]


def _im2col(x, stride, ho):
    n, _, _, c = x.shape
    span = stride * (ho - 1) + 1
    taps = []
    for dy in range(3):
        for dx in range(3):
            taps.append(x[:, dy:dy + span:stride, dx:dx + span:stride, :])
    p = jnp.stack(taps, axis=3)
    return p.reshape(n * ho * ho, 9 * c)


def _wflat(w):
    return jnp.transpose(w, (2, 3, 1, 0)).reshape(-1, w.shape[0])


def _conv_relu(x, w, b, stride, ho):
    n = x.shape[0]
    patches = _im2col(x, stride, ho)
    out = _mm(patches, _wflat(w), b, relu=True)
    return out.reshape(n, ho, ho, w.shape[0])


def _mm9_body(*refs, relu, c):
    a_refs = refs[:9]
    w_ref, b_ref, o_ref = refs[9:]
    acc = jnp.zeros(o_ref.shape, jnp.float32)
    for t in range(9):
        acc = acc + jnp.dot(a_refs[t][...], w_ref[t * c:(t + 1) * c, :],
                            preferred_element_type=jnp.float32)
    acc = acc + b_ref[...]
    if relu:
        acc = jnp.maximum(acc, 0.0)
    o_ref[...] = acc


def _conv_relu9(x, w, b, stride, ho, bm=512):
    n, _, _, c = x.shape
    o = w.shape[0]
    span = stride * (ho - 1) + 1
    m = n * ho * ho
    mp = -(-m // bm) * bm
    slices = []
    for dy in range(3):
        for dx in range(3):
            s = x[:, dy:dy + span:stride, dx:dx + span:stride, :]
            slices.append(jnp.pad(s.reshape(m, c), ((0, mp - m), (0, 0))))
    bn = 128 if o > 128 else o
    out = pl.pallas_call(
        functools.partial(_mm9_body, relu=True, c=c),
        grid=(mp // bm, o // bn),
        in_specs=[pl.BlockSpec((bm, c), lambda i, j: (i, 0))] * 9
        + [pl.BlockSpec((9 * c, bn), lambda i, j: (0, j)),
           pl.BlockSpec((1, bn), lambda i, j: (0, j))],
        out_specs=pl.BlockSpec((bm, bn), lambda i, j: (i, j)),
        out_shape=jax.ShapeDtypeStruct((mp, o), jnp.float32),
    )(*slices, _wflat(w), b.reshape(1, o))
    return out[:m].reshape(n, ho, ho, o)


def _encoder(x_nchw, p, pre):
    x = jnp.transpose(x_nchw, (0, 2, 3, 1))
    x = _conv_relu(x, p[pre + '_w1'], p[pre + '_b1'], 2, 139)
    x = _conv_relu9(x, p[pre + '_w2'], p[pre + '_b2'], 1, 137)
    x = _conv_relu9(x, p[pre + '_w3'], p[pre + '_b3'], 2, 68)
    return jnp.transpose(x, (0, 3, 1, 2))


def _affine_grid(theta, hw):
    h = w = hw
    xs = (2.0 * jnp.arange(w, dtype=jnp.float32) + 1.0) / w - 1.0
    ys = (2.0 * jnp.arange(h, dtype=jnp.float32) + 1.0) / h - 1.0
    gx, gy = jnp.meshgrid(xs, ys)
    base = jnp.stack([gx, gy, jnp.ones_like(gx)], axis=-1)
    return jnp.einsum('nij,hwj->nhwi', theta, base)


def _grid_sample(im, grid):
    n, c, h, w = im.shape
    ix = ((grid[..., 0] + 1.0) * w - 1.0) / 2.0
    iy = ((grid[..., 1] + 1.0) * h - 1.0) / 2.0
    ix0 = jnp.floor(ix)
    iy0 = jnp.floor(iy)
    ix1 = ix0 + 1.0
    iy1 = iy0 + 1.0
    wx1 = ix - ix0
    wx0 = 1.0 - wx1
    wy1 = iy - iy0
    wy0 = 1.0 - wy1
    bidx = jnp.arange(n)[:, None, None]

    def g(xc, yc):
        xi = jnp.clip(xc, 0, w - 1).astype(jnp.int32)
        yi = jnp.clip(yc, 0, h - 1).astype(jnp.int32)
        valid = ((xc >= 0) & (xc <= w - 1) & (yc >= 0) & (yc <= h - 1))
        return im[bidx, :, yi, xi] * valid[..., None].astype(im.dtype)

    out = (g(ix0, iy0) * (wx0 * wy0)[..., None]
           + g(ix1, iy0) * (wx1 * wy0)[..., None]
           + g(ix0, iy1) * (wx0 * wy1)[..., None]
           + g(ix1, iy1) * (wx1 * wy1)[..., None])
    return jnp.transpose(out, (0, 3, 1, 2))


def _head_body(xw_ref, sp_ref, g1b, w2, b2, w3, b3, m1w, m1b, m2w, m2b,
               f1a, f1b_, f1bias, f2w, f2b, f3w, f3b, o_ref):
    xw = xw_ref[...]
    m = jnp.maximum(jnp.maximum(xw[0:4], xw[4:8]),
                    jnp.maximum(xw[8:12], xw[12:16]))
    z = jnp.maximum(0.25 * m + g1b[...], 0.0)
    z = jnp.maximum(0.25 * jnp.dot(z, w2[...],
                                   preferred_element_type=jnp.float32)
                    + b2[...], 0.0)
    z = 0.25 * jnp.dot(z, w3[...], preferred_element_type=jnp.float32) + b3[...]
    mt = jnp.maximum(sp_ref[...] * m1w[...] + m1b[...], 0.0)
    mt = jnp.maximum(jnp.dot(mt, m2w[...],
                             preferred_element_type=jnp.float32) + m2b[...], 0.0)
    h = jnp.maximum(jnp.dot(z, f1a[...], preferred_element_type=jnp.float32)
                    + jnp.dot(mt, f1b_[...], preferred_element_type=jnp.float32)
                    + f1bias[...], 0.0)
    h = jnp.maximum(jnp.dot(h, f2w[...], preferred_element_type=jnp.float32)
                    + f2b[...], 0.0)
    o_ref[...] = (jnp.dot(h, f3w[...], preferred_element_type=jnp.float32)
                  + f3b[...])


def _head(xw, speed, p):
    f1a = p['f1_w'][:512]
    f1b_ = p['f1_w'][512:]
    f3w = jnp.pad(p['f3_w'], ((0, 0), (0, 125)))
    f3b = jnp.pad(p['f3_b'], (0, 125))
    args = [xw, speed.reshape(4, 1),
            p['g1_b'].reshape(1, -1),
            p['g2_w'], p['g2_b'].reshape(1, -1),
            p['g3_w'], p['g3_b'].reshape(1, -1),
            p['m1_w'], p['m1_b'].reshape(1, -1),
            p['m2_w'], p['m2_b'].reshape(1, -1),
            f1a, f1b_, p['f1_b'].reshape(1, -1),
            p['f2_w'], p['f2_b'].reshape(1, -1),
            f3w, f3b.reshape(1, -1)]
    out = pl.pallas_call(
        _head_body,
        out_shape=jax.ShapeDtypeStruct((4, 128), jnp.float32),
    )(*args)
    return out[:, :3]


def kernel(ego_lidar, ego_speed, other_lidar, other_transform, params):
    B = ego_lidar.shape[0]
    ego_rep = _encoder(ego_lidar, params, 'e1')
    ol = jnp.transpose(other_lidar, (1, 0, 2, 3, 4)).reshape(
        3 * B, *other_lidar.shape[2:])
    other_rep = _encoder(ol, params, 'e2')

    sel = jnp.array([0, 1, 3])
    feats = [ego_rep.reshape(B, -1)]
    for i in range(3):
        rep = other_rep[i * B:(i + 1) * B]
        theta = other_transform[:, i][:, :2, :][:, :, sel]
        grid = _affine_grid(theta, 68)
        warped = _grid_sample(rep, grid)
        feats.append(warped.reshape(B, -1))
    x_all = jnp.concatenate(feats, axis=0)

    zeros512 = jnp.zeros((512,), jnp.float32)
    xw = _mm(x_all, params['g1_w'], zeros512, relu=False)

    pc = _head(xw, ego_speed, params)
    return (pc[:, 0], pc[:, 1], pc[:, 2])
